# Initial kernel scaffold; baseline (speedup 1.0000x reference)
import jax
import jax.numpy as jnp
from jax.experimental import pallas as pl


def kernel(tokens, segment, token_table, pos_table, sent_table):
    raise NotImplementedError("write your pallas kernel here")


# speedup vs baseline: 4.0038x; 4.0038x over previous
import functools

import jax
import jax.numpy as jnp
from jax import lax
from jax.experimental import pallas as pl
from jax.experimental.pallas import tpu as pltpu
from jax.experimental.pallas import tpu_sc as plsc

NC = 2
NS = 16
LANES = 16
CH = 128


def _sc_embed(tok, seg, table, comb, *, n_rows, d, n_pos):
    n_workers = NC * NS
    rows_per_worker = n_rows // n_workers
    n_chunks = rows_per_worker // CH
    mesh = plsc.VectorSubcoreMesh(
        core_axis_name="c", subcore_axis_name="s",
        num_cores=NC, num_subcores=NS)

    @functools.partial(
        pl.kernel,
        out_type=jax.ShapeDtypeStruct((n_rows, d), jnp.float32),
        mesh=mesh,
        scratch_types=dict(
            tok_v=pltpu.VMEM((CH,), jnp.int32),
            seg_v=pltpu.VMEM((CH,), jnp.int32),
            cidx_v=pltpu.VMEM((CH,), jnp.int32),
            rows_v=pltpu.VMEM((CH, d), jnp.float32),
            add_v=pltpu.VMEM((CH, d), jnp.float32),
            sem_a=pltpu.SemaphoreType.DMA,
            sem_b=pltpu.SemaphoreType.DMA,
        ),
        compiler_params=pltpu.CompilerParams(use_tc_tiling_on_sc=False),
    )
    def k(tok_hbm, seg_hbm, table_hbm, comb_hbm, out_hbm,
          tok_v, seg_v, cidx_v, rows_v, add_v, sem_a, sem_b):
        wid = lax.axis_index("s") * NC + lax.axis_index("c")
        wbase = wid * rows_per_worker

        def chunk(kk, carry):
            base = wbase + kk * CH
            pltpu.sync_copy(tok_hbm.at[pl.ds(base, CH)], tok_v)
            pltpu.sync_copy(seg_hbm.at[pl.ds(base, CH)], seg_v)
            for j in range(CH // LANES):
                s16 = seg_v[pl.ds(j * LANES, LANES)]
                flat = base + j * LANES + lax.iota(jnp.int32, LANES)
                cidx_v[pl.ds(j * LANES, LANES)] = (
                    s16 * n_pos + lax.rem(flat, n_pos))
            g1 = pltpu.async_copy(table_hbm.at[tok_v], rows_v, sem_a)
            g2 = pltpu.async_copy(comb_hbm.at[cidx_v], add_v, sem_b)
            g1.wait()
            g2.wait()

            def addrow(r, c):
                for dd in range(d // LANES):
                    sl = pl.ds(dd * LANES, LANES)
                    rows_v[r, sl] = rows_v[r, sl] + add_v[r, sl]
                return c

            lax.fori_loop(0, CH, addrow, 0)
            pltpu.sync_copy(rows_v, out_hbm.at[pl.ds(base, CH)])
            return carry

        lax.fori_loop(0, n_chunks, chunk, 0)

    return k(tok, seg, table, comb)


def kernel(tokens, segment, token_table, pos_table, sent_table):
    b, l = tokens.shape
    v, d = token_table.shape
    n_sent = sent_table.shape[0]
    tok = tokens.reshape(-1).astype(jnp.int32)
    seg = segment.reshape(-1).astype(jnp.int32)
    comb = (sent_table[:, None, :] + pos_table[None, :, :]).reshape(
        n_sent * l, d)
    out = _sc_embed(tok, seg, token_table, comb,
                    n_rows=b * l, d=d, n_pos=l)
    return out.reshape(b, l, d)


# speedup vs baseline: 4.3309x; 1.0817x over previous
import functools

import jax
import jax.numpy as jnp
from jax import lax
from jax.experimental import pallas as pl
from jax.experimental.pallas import tpu as pltpu
from jax.experimental.pallas import tpu_sc as plsc

NC = 2
NS = 16
LANES = 16
CH = 128


def _sc_embed(tok, seg, table, comb, *, n_rows, d, n_pos):
    n_workers = NC * NS
    rows_per_worker = n_rows // n_workers
    n_chunks = rows_per_worker // CH
    mesh = plsc.VectorSubcoreMesh(
        core_axis_name="c", subcore_axis_name="s",
        num_cores=NC, num_subcores=NS)

    @functools.partial(
        pl.kernel,
        out_type=jax.ShapeDtypeStruct((n_rows, d), jnp.float32),
        mesh=mesh,
        scratch_types=dict(
            tok_v=pltpu.VMEM((CH,), jnp.int32),
            seg_v=pltpu.VMEM((CH,), jnp.int32),
            cidx_v=pltpu.VMEM((CH,), jnp.int32),
            rows_v=pltpu.VMEM((CH, d), jnp.float32),
            comb_sh=pltpu.VMEM_SHARED((2 * 200, 64), jnp.float32),
            sem_a=pltpu.SemaphoreType.DMA,
            sem_b=pltpu.SemaphoreType.DMA,
        ),
        compiler_params=pltpu.CompilerParams(use_tc_tiling_on_sc=False),
    )
    def k(tok_hbm, seg_hbm, table_hbm, comb_hbm, out_hbm,
          tok_v, seg_v, cidx_v, rows_v, comb_sh, sem_a, sem_b):
        wid = lax.axis_index("s") * NC + lax.axis_index("c")
        wbase = wid * rows_per_worker

        @pl.when(lax.axis_index("s") == 0)
        def _():
            pltpu.sync_copy(comb_hbm, comb_sh)

        plsc.subcore_barrier()

        def chunk(kk, carry):
            base = wbase + kk * CH
            pltpu.sync_copy(tok_hbm.at[pl.ds(base, CH)], tok_v)
            pltpu.sync_copy(seg_hbm.at[pl.ds(base, CH)], seg_v)
            for j in range(CH // LANES):
                s16 = seg_v[pl.ds(j * LANES, LANES)]
                flat = base + j * LANES + lax.iota(jnp.int32, LANES)
                cidx_v[pl.ds(j * LANES, LANES)] = (
                    s16 * n_pos + lax.rem(flat, n_pos))
            pltpu.async_copy(table_hbm.at[tok_v], rows_v, sem_a).wait()
            pltpu.async_copy(comb_sh.at[cidx_v], rows_v, sem_b,
                             add=True).wait()
            pltpu.sync_copy(rows_v, out_hbm.at[pl.ds(base, CH)])
            return carry

        lax.fori_loop(0, n_chunks, chunk, 0)

    return k(tok, seg, table, comb)


def kernel(tokens, segment, token_table, pos_table, sent_table):
    b, l = tokens.shape
    v, d = token_table.shape
    n_sent = sent_table.shape[0]
    tok = tokens.reshape(-1).astype(jnp.int32)
    seg = segment.reshape(-1).astype(jnp.int32)
    comb = (sent_table[:, None, :] + pos_table[None, :, :]).reshape(
        n_sent * l, d)
    out = _sc_embed(tok, seg, token_table, comb,
                    n_rows=b * l, d=d, n_pos=l)
    return out.reshape(b, l, d)
